# baseline (speedup 1.0000x reference)
import functools

import jax
import jax.numpy as jnp
from jax import lax
from jax.experimental import pallas as pl
from jax.experimental.pallas import tpu as pltpu
from jax.experimental.pallas import tpu_sc as plsc


def _make_sc_gather(num_rows, emb_dim, dtype, chunk=128):
    info = plsc.get_sparse_core_info()
    nw = info.num_cores * info.num_subcores
    assert num_rows % (nw * chunk) == 0
    rows_per_w = num_rows // nw
    chunks_per_w = rows_per_w // chunk
    mesh = plsc.VectorSubcoreMesh(core_axis_name="c", subcore_axis_name="s")

    @functools.partial(
        pl.kernel,
        mesh=mesh,
        out_type=jax.ShapeDtypeStruct((num_rows, emb_dim), dtype),
        compiler_params=pltpu.CompilerParams(use_tc_tiling_on_sc=False),
        scratch_types=[
            pltpu.VMEM((chunks_per_w, chunk), jnp.int32),
            pltpu.VMEM((2, chunk, emb_dim), dtype),
            pltpu.SemaphoreType.DMA,
            pltpu.SemaphoreType.DMA,
        ],
    )
    def gather_k(idx_hbm, table_hbm, out_hbm, idx_v, rows_v, gsem, osem):
        wid = lax.axis_index("s") * info.num_cores + lax.axis_index("c")
        row0 = wid * chunks_per_w
        pltpu.sync_copy(idx_hbm.at[wid], idx_v)

        def gather_start(j, buf):
            pltpu.async_copy(table_hbm.at[idx_v.at[j]], rows_v.at[buf], gsem)

        def gather_wait(j, buf):
            pltpu.make_async_copy(
                table_hbm.at[idx_v.at[j]], rows_v.at[buf], gsem
            ).wait()

        def out_start(j, buf):
            pltpu.async_copy(
                rows_v.at[buf],
                out_hbm.at[pl.ds((row0 + j) * chunk, chunk)],
                osem,
            )

        def out_wait(j, buf):
            pltpu.make_async_copy(
                rows_v.at[buf],
                out_hbm.at[pl.ds((row0 + j) * chunk, chunk)],
                osem,
            ).wait()

        gather_start(0, 0)

        def body(j, _):
            buf = lax.rem(j, 2)
            gather_wait(j, buf)

            @pl.when(j + 1 < chunks_per_w)
            def _():
                gather_start(j + 1, 1 - buf)

            @pl.when(j >= 2)
            def _():
                out_wait(j - 2, buf)

            out_start(j, buf)
            return 0

        lax.fori_loop(0, chunks_per_w, body, 0)
        out_wait(chunks_per_w - 2, lax.rem(chunks_per_w - 2, 2))
        out_wait(chunks_per_w - 1, lax.rem(chunks_per_w - 1, 2))

    def gather_fn(idx_flat, table):
        return gather_k(idx_flat.reshape(nw, chunks_per_w, chunk), table)

    return gather_fn


def _gru_body(emb_ref, wih_ref, whh_ref, bias_a_ref, bias_n_ref, hs_ref,
              hn_ref, h_ref):
    t_len = pl.num_programs(0)
    t = pl.program_id(0)
    h_dim = hn_ref.shape[0]

    @pl.when(t == 0)
    def _():
        h_ref[...] = jnp.zeros_like(h_ref)

    x = emb_ref[0]
    h = h_ref[...]
    gx = jnp.dot(x, wih_ref[...], preferred_element_type=jnp.float32)
    gh = jnp.dot(h, whh_ref[...], preferred_element_type=jnp.float32)
    gh = gh + bias_a_ref[...]
    rz = jax.nn.sigmoid(gx[:, : 2 * h_dim] + gh[:, : 2 * h_dim])
    z = rz[:, h_dim:]
    n = jnp.tanh(
        gx[:, 2 * h_dim:] + rz[:, :h_dim] * gh[:, 2 * h_dim:]
        + bias_n_ref[...]
    )
    h = (1.0 - z) * n + z * h
    h_ref[...] = h
    hs_ref[0] = h.T

    @pl.when(t == t_len - 1)
    def _():
        hn_ref[...] = h.T


def _gru(emb, wih_t, whh_t, bias_a, bias_n, interpret=False):
    t_len, b, e = emb.shape
    h_dim = whh_t.shape[0]
    return pl.pallas_call(
        _gru_body,
        grid=(t_len,),
        in_specs=[
            pl.BlockSpec((1, b, e), lambda t: (t, 0, 0)),
            pl.BlockSpec((e, 3 * h_dim), lambda t: (0, 0)),
            pl.BlockSpec((h_dim, 3 * h_dim), lambda t: (0, 0)),
            pl.BlockSpec((1, 3 * h_dim), lambda t: (0, 0)),
            pl.BlockSpec((1, h_dim), lambda t: (0, 0)),
        ],
        out_specs=[
            pl.BlockSpec((1, h_dim, b), lambda t: (t, 0, 0)),
            pl.BlockSpec((h_dim, b), lambda t: (0, 0)),
        ],
        out_shape=[
            jax.ShapeDtypeStruct((t_len, h_dim, b), jnp.float32),
            jax.ShapeDtypeStruct((h_dim, b), jnp.float32),
        ],
        scratch_shapes=[pltpu.VMEM((b, h_dim), jnp.float32)],
        compiler_params=pltpu.CompilerParams(
            fuse_transposed_lhs_in_matmul=False,
        ),
        interpret=interpret,
    )(emb, wih_t, whh_t, bias_a, bias_n)


def kernel(data, emb_table, W_ih, W_hh, b_ih, b_hh):
    b, t_len = data.shape
    e = emb_table.shape[1]
    h_dim = W_hh.shape[1]
    num_rows = b * t_len
    chunk = 128

    e_pad = 128
    table128 = jnp.pad(
        emb_table.astype(jnp.bfloat16), ((0, 0), (0, e_pad - e))
    )
    idx_flat = data.T.reshape(num_rows)
    gather = _make_sc_gather(num_rows, e_pad, jnp.bfloat16, chunk=chunk)
    emb_flat = gather(idx_flat, table128)
    emb = emb_flat.reshape(t_len, b, e_pad)

    bias_a = jnp.concatenate(
        [b_ih[: 2 * h_dim] + b_hh[: 2 * h_dim], b_hh[2 * h_dim:]]
    ).reshape(1, 3 * h_dim)
    bias_n = b_ih[2 * h_dim:].reshape(1, h_dim)
    wih_pad = jnp.pad(W_ih.T, ((0, e_pad - e), (0, 0))).astype(jnp.bfloat16)
    hs_thb, hn_hb = _gru(emb, wih_pad, W_hh.T, bias_a, bias_n)
    hidden_states = jnp.transpose(hs_thb, (2, 0, 1))
    final_h = hn_hb.T[None]
    return hidden_states, final_h


# speedup vs baseline: 2.1907x; 2.1907x over previous
import functools

import jax
import jax.numpy as jnp
from jax import lax
from jax.experimental import pallas as pl
from jax.experimental.pallas import tpu as pltpu
from jax.experimental.pallas import tpu_sc as plsc


def _make_sc_gather(num_rows, emb_dim, dtype, chunk=128):
    info = plsc.get_sparse_core_info()
    nw = info.num_cores * info.num_subcores
    assert num_rows % (nw * chunk) == 0
    rows_per_w = num_rows // nw
    chunks_per_w = rows_per_w // chunk
    mesh = plsc.VectorSubcoreMesh(core_axis_name="c", subcore_axis_name="s")

    @functools.partial(
        pl.kernel,
        mesh=mesh,
        out_type=jax.ShapeDtypeStruct((num_rows, emb_dim), dtype),
        compiler_params=pltpu.CompilerParams(use_tc_tiling_on_sc=False),
        scratch_types=[
            pltpu.VMEM((chunks_per_w, chunk), jnp.int32),
            pltpu.VMEM((2, chunk, emb_dim), dtype),
            pltpu.SemaphoreType.DMA,
            pltpu.SemaphoreType.DMA,
        ],
    )
    def gather_k(idx_hbm, table_hbm, out_hbm, idx_v, rows_v, gsem, osem):
        wid = lax.axis_index("s") * info.num_cores + lax.axis_index("c")
        row0 = wid * chunks_per_w
        pltpu.sync_copy(idx_hbm.at[wid], idx_v)

        def gather_start(j, buf):
            pltpu.async_copy(table_hbm.at[idx_v.at[j]], rows_v.at[buf], gsem)

        def gather_wait(j, buf):
            pltpu.make_async_copy(
                table_hbm.at[idx_v.at[j]], rows_v.at[buf], gsem
            ).wait()

        def out_start(j, buf):
            pltpu.async_copy(
                rows_v.at[buf],
                out_hbm.at[pl.ds((row0 + j) * chunk, chunk)],
                osem,
            )

        def out_wait(j, buf):
            pltpu.make_async_copy(
                rows_v.at[buf],
                out_hbm.at[pl.ds((row0 + j) * chunk, chunk)],
                osem,
            ).wait()

        gather_start(0, 0)

        def body(j, _):
            buf = lax.rem(j, 2)
            gather_wait(j, buf)

            @pl.when(j + 1 < chunks_per_w)
            def _():
                gather_start(j + 1, 1 - buf)

            @pl.when(j >= 2)
            def _():
                out_wait(j - 2, buf)

            out_start(j, buf)
            return 0

        lax.fori_loop(0, chunks_per_w, body, 0)
        out_wait(chunks_per_w - 2, lax.rem(chunks_per_w - 2, 2))
        out_wait(chunks_per_w - 1, lax.rem(chunks_per_w - 1, 2))

    def gather_fn(idx_flat, table):
        return gather_k(idx_flat.reshape(nw, chunks_per_w, chunk), table)

    return gather_fn


def _gru_body(emb_ref, wih_ref, whh_ref, bias_a_ref, bias_n_ref, hs_ref,
              hn_ref, h_ref):
    t_len = pl.num_programs(0)
    t = pl.program_id(0)
    h_dim = hn_ref.shape[0]

    @pl.when(t == 0)
    def _():
        h_ref[...] = jnp.zeros_like(h_ref)

    x = emb_ref[0]
    h = h_ref[...]
    gx = jnp.dot(x, wih_ref[...], preferred_element_type=jnp.float32)
    gh = jnp.dot(h, whh_ref[...], preferred_element_type=jnp.float32)
    gh = gh + bias_a_ref[...]
    rz = jax.nn.sigmoid(gx[:, : 2 * h_dim] + gh[:, : 2 * h_dim])
    z = rz[:, h_dim:]
    n = jnp.tanh(
        gx[:, 2 * h_dim:] + rz[:, :h_dim] * gh[:, 2 * h_dim:]
        + bias_n_ref[...]
    )
    h = (1.0 - z) * n + z * h
    h_ref[...] = h
    hs_ref[0] = h.T

    @pl.when(t == t_len - 1)
    def _():
        hn_ref[...] = h.T


def _gru(emb, wih_t, whh_t, bias_a, bias_n, interpret=False):
    t_len, b, e = emb.shape
    h_dim = whh_t.shape[0]
    return pl.pallas_call(
        _gru_body,
        grid=(t_len,),
        in_specs=[
            pl.BlockSpec((1, b, e), lambda t: (t, 0, 0)),
            pl.BlockSpec((e, 3 * h_dim), lambda t: (0, 0)),
            pl.BlockSpec((h_dim, 3 * h_dim), lambda t: (0, 0)),
            pl.BlockSpec((1, 3 * h_dim), lambda t: (0, 0)),
            pl.BlockSpec((1, h_dim), lambda t: (0, 0)),
        ],
        out_specs=[
            pl.BlockSpec((1, h_dim, b), lambda t: (t, 0, 0)),
            pl.BlockSpec((h_dim, b), lambda t: (0, 0)),
        ],
        out_shape=[
            jax.ShapeDtypeStruct((t_len, h_dim, b), jnp.float32),
            jax.ShapeDtypeStruct((h_dim, b), jnp.float32),
        ],
        scratch_shapes=[pltpu.VMEM((b, h_dim), jnp.float32)],
        compiler_params=pltpu.CompilerParams(
            fuse_transposed_lhs_in_matmul=False,
        ),
        interpret=interpret,
    )(emb, wih_t, whh_t, bias_a, bias_n)


def kernel(data, emb_table, W_ih, W_hh, b_ih, b_hh):
    b, t_len = data.shape
    e = emb_table.shape[1]
    h_dim = W_hh.shape[1]
    num_rows = b * t_len
    chunk = 128

    e_pad = 128
    table128 = jnp.pad(emb_table, ((0, 0), (0, e_pad - e)))
    idx_flat = data.T.reshape(num_rows)
    gather = _make_sc_gather(num_rows, e_pad, jnp.float32, chunk=chunk)
    emb_flat = gather(idx_flat, table128)
    emb = emb_flat.reshape(t_len, b, e_pad)

    bias_a = jnp.concatenate(
        [b_ih[: 2 * h_dim] + b_hh[: 2 * h_dim], b_hh[2 * h_dim:]]
    ).reshape(1, 3 * h_dim)
    bias_n = b_ih[2 * h_dim:].reshape(1, h_dim)
    wih_pad = jnp.pad(W_ih.T, ((0, e_pad - e), (0, 0)))
    hs_thb, hn_hb = _gru(emb, wih_pad, W_hh.T, bias_a, bias_n)
    hidden_states = jnp.transpose(hs_thb, (2, 0, 1))
    final_h = hn_hb.T[None]
    return hidden_states, final_h


# speedup vs baseline: 2.2475x; 1.0260x over previous
import functools

import jax
import jax.numpy as jnp
from jax import lax
from jax.experimental import pallas as pl
from jax.experimental.pallas import tpu as pltpu
from jax.experimental.pallas import tpu_sc as plsc


def _make_sc_gather(num_rows, emb_dim, dtype, chunk=128):
    info = plsc.get_sparse_core_info()
    nw = info.num_cores * info.num_subcores
    assert num_rows % (nw * chunk) == 0
    rows_per_w = num_rows // nw
    chunks_per_w = rows_per_w // chunk
    mesh = plsc.VectorSubcoreMesh(core_axis_name="c", subcore_axis_name="s")

    @functools.partial(
        pl.kernel,
        mesh=mesh,
        out_type=jax.ShapeDtypeStruct((num_rows, emb_dim), dtype),
        compiler_params=pltpu.CompilerParams(use_tc_tiling_on_sc=False),
        scratch_types=[
            pltpu.VMEM((chunks_per_w, chunk), jnp.int32),
            pltpu.VMEM((4, chunk, emb_dim), dtype),
            pltpu.SemaphoreType.DMA,
            pltpu.SemaphoreType.DMA,
            pltpu.SemaphoreType.DMA,
            pltpu.SemaphoreType.DMA,
        ],
    )
    def gather_k(idx_hbm, table_hbm, out_hbm, idx_v, rows_v,
                 gsem0, gsem1, osem0, osem1):
        wid = lax.axis_index("s") * info.num_cores + lax.axis_index("c")
        row0 = wid * chunks_per_w
        pltpu.sync_copy(idx_hbm.at[wid], idx_v)

        def gather_start(j, buf, sem):
            pltpu.async_copy(table_hbm.at[idx_v.at[j]], rows_v.at[buf], sem)

        def gather_wait(j, buf, sem):
            pltpu.make_async_copy(
                table_hbm.at[idx_v.at[j]], rows_v.at[buf], sem
            ).wait()

        def out_start(j, buf, sem):
            pltpu.async_copy(
                rows_v.at[buf],
                out_hbm.at[pl.ds((row0 + j) * chunk, chunk)],
                sem,
            )

        def out_wait(j, buf, sem):
            pltpu.make_async_copy(
                rows_v.at[buf],
                out_hbm.at[pl.ds((row0 + j) * chunk, chunk)],
                sem,
            ).wait()

        gather_start(0, 0, gsem0)
        gather_start(1, 1, gsem1)

        def body(j, _):
            buf = lax.rem(j, 4)
            gsel = lax.rem(j, 2)

            @pl.when(gsel == 0)
            def _():
                gather_wait(j, buf, gsem0)

            @pl.when(gsel == 1)
            def _():
                gather_wait(j, buf, gsem1)

            @pl.when(j >= 2)
            def _():
                @pl.when(gsel == 0)
                def _():
                    out_wait(j - 2, lax.rem(j - 2, 4), osem0)

                @pl.when(gsel == 1)
                def _():
                    out_wait(j - 2, lax.rem(j - 2, 4), osem1)

            @pl.when(j + 2 < chunks_per_w)
            def _():
                @pl.when(gsel == 0)
                def _():
                    gather_start(j + 2, lax.rem(j + 2, 4), gsem0)

                @pl.when(gsel == 1)
                def _():
                    gather_start(j + 2, lax.rem(j + 2, 4), gsem1)

            @pl.when(gsel == 0)
            def _():
                out_start(j, buf, osem0)

            @pl.when(gsel == 1)
            def _():
                out_start(j, buf, osem1)

            return 0

        lax.fori_loop(0, chunks_per_w, body, 0)
        j0 = chunks_per_w - 2
        out_wait(j0, lax.rem(j0, 4), [osem0, osem1][j0 % 2])
        j1 = chunks_per_w - 1
        out_wait(j1, lax.rem(j1, 4), [osem0, osem1][j1 % 2])

    def gather_fn(idx_flat, table):
        return gather_k(idx_flat.reshape(nw, chunks_per_w, chunk), table)

    return gather_fn


def _gru_body(emb_ref, wih_ref, whh_ref, bias_a_ref, bias_n_ref, hs_ref,
              hn_ref, h_ref):
    t_len = pl.num_programs(0)
    t = pl.program_id(0)
    h_dim = hn_ref.shape[0]

    @pl.when(t == 0)
    def _():
        h_ref[...] = jnp.zeros_like(h_ref)

    x = emb_ref[0]
    h = h_ref[...]
    gx = jnp.dot(x, wih_ref[...], preferred_element_type=jnp.float32)
    gh = jnp.dot(h, whh_ref[...], preferred_element_type=jnp.float32)
    gh = gh + bias_a_ref[...]
    rz = jax.nn.sigmoid(gx[:, : 2 * h_dim] + gh[:, : 2 * h_dim])
    z = rz[:, h_dim:]
    n = jnp.tanh(
        gx[:, 2 * h_dim:] + rz[:, :h_dim] * gh[:, 2 * h_dim:]
        + bias_n_ref[...]
    )
    h = (1.0 - z) * n + z * h
    h_ref[...] = h
    hs_ref[0] = h.T

    @pl.when(t == t_len - 1)
    def _():
        hn_ref[...] = h.T


def _gru(emb, wih_t, whh_t, bias_a, bias_n, interpret=False):
    t_len, b, e = emb.shape
    h_dim = whh_t.shape[0]
    return pl.pallas_call(
        _gru_body,
        grid=(t_len,),
        in_specs=[
            pl.BlockSpec((1, b, e), lambda t: (t, 0, 0)),
            pl.BlockSpec((e, 3 * h_dim), lambda t: (0, 0)),
            pl.BlockSpec((h_dim, 3 * h_dim), lambda t: (0, 0)),
            pl.BlockSpec((1, 3 * h_dim), lambda t: (0, 0)),
            pl.BlockSpec((1, h_dim), lambda t: (0, 0)),
        ],
        out_specs=[
            pl.BlockSpec((1, h_dim, b), lambda t: (t, 0, 0)),
            pl.BlockSpec((h_dim, b), lambda t: (0, 0)),
        ],
        out_shape=[
            jax.ShapeDtypeStruct((t_len, h_dim, b), jnp.float32),
            jax.ShapeDtypeStruct((h_dim, b), jnp.float32),
        ],
        scratch_shapes=[pltpu.VMEM((b, h_dim), jnp.float32)],
        compiler_params=pltpu.CompilerParams(
            fuse_transposed_lhs_in_matmul=False,
        ),
        interpret=interpret,
    )(emb, wih_t, whh_t, bias_a, bias_n)


def kernel(data, emb_table, W_ih, W_hh, b_ih, b_hh):
    b, t_len = data.shape
    e = emb_table.shape[1]
    h_dim = W_hh.shape[1]
    num_rows = b * t_len
    chunk = 128

    e_pad = 128
    table128 = jnp.pad(emb_table, ((0, 0), (0, e_pad - e)))
    idx_flat = data.T.reshape(num_rows)
    gather = _make_sc_gather(num_rows, e_pad, jnp.float32, chunk=chunk)
    emb_flat = gather(idx_flat, table128)
    emb = emb_flat.reshape(t_len, b, e_pad)

    bias_a = jnp.concatenate(
        [b_ih[: 2 * h_dim] + b_hh[: 2 * h_dim], b_hh[2 * h_dim:]]
    ).reshape(1, 3 * h_dim)
    bias_n = b_ih[2 * h_dim:].reshape(1, h_dim)
    wih_pad = jnp.pad(W_ih.T, ((0, e_pad - e), (0, 0)))
    hs_thb, hn_hb = _gru(emb, wih_pad, W_hh.T, bias_a, bias_n)
    hidden_states = jnp.transpose(hs_thb, (2, 0, 1))
    final_h = hn_hb.T[None]
    return hidden_states, final_h
